# baseline (speedup 1.0000x reference)
import functools

import jax
import jax.numpy as jnp
from jax import lax
from jax.experimental import pallas as pl
from jax.experimental.pallas import tpu as pltpu
from jax.experimental.pallas import tpu_sc as plsc

NC = 2
NS = 16
L = 16
NW = NC * NS

BATCH = 16384
DIM = 32
BPW = BATCH // NW
NG = BPW // L
CH = 64
NCHUNK = BPW // CH


def _make_sc_kernel():
  mesh = plsc.VectorSubcoreMesh(core_axis_name="c", subcore_axis_name="s",
                                num_cores=NC, num_subcores=NS)

  @functools.partial(
      pl.kernel,
      out_type=jax.ShapeDtypeStruct((BATCH,), jnp.float32),
      mesh=mesh,
      scratch_types=[
          pltpu.VMEM((BPW,), jnp.int32),
          pltpu.VMEM((BPW,), jnp.int32),
          pltpu.VMEM_SHARED((NS, BPW, DIM), jnp.float32),
          pltpu.VMEM_SHARED((NS, BPW, DIM), jnp.float32),
          pltpu.VMEM((2, CH, DIM), jnp.float32),
          pltpu.VMEM((2, CH, DIM), jnp.float32),
          pltpu.VMEM((48,), jnp.float32),
          pltpu.VMEM((BPW,), jnp.float32),
          pltpu.SemaphoreType.DMA,
          pltpu.SemaphoreType.DMA,
          pltpu.SemaphoreType.DMA,
          pltpu.SemaphoreType.DMA,
          pltpu.SemaphoreType.DMA,
          pltpu.SemaphoreType.DMA,
      ],
      compiler_params=pltpu.CompilerParams(needs_layout_passes=False),
  )
  def sc_kernel(user_hbm, item_hbm, wb_hbm, utab_hbm, itab_hbm, out_hbm,
                uidx_v, iidx_v, uspm, ispm, ubuf_v, ibuf_v, wb_v, out_v,
                sem_u, sem_i, sem_cu0, sem_ci0, sem_cu1, sem_ci1):
    sid = lax.axis_index("s")
    wid = sid * NC + lax.axis_index("c")
    base = wid * BPW

    pltpu.sync_copy(wb_hbm, wb_v)
    pltpu.sync_copy(user_hbm.at[pl.ds(base, BPW)], uidx_v)
    pltpu.sync_copy(item_hbm.at[pl.ds(base, BPW)], iidx_v)

    lane = lax.iota(jnp.int32, L)
    bias = wb_v[pl.ds(DIM, L)]
    w0 = wb_v[pl.ds(0, L)]
    w1 = wb_v[pl.ds(L, L)]

    def fire(g, carry):
      uvec = uidx_v[pl.ds(g * L, L)]
      ivec = iidx_v[pl.ds(g * L, L)]
      for j in range(L):
        pltpu.async_copy(utab_hbm.at[pl.ds(uvec[j], 1)],
                         uspm.at[sid, pl.ds(g * L + j, 1)], sem_u)
        pltpu.async_copy(itab_hbm.at[pl.ds(ivec[j], 1)],
                         ispm.at[sid, pl.ds(g * L + j, 1)], sem_i)
      return carry

    lax.fori_loop(0, NG, fire, 0)
    pltpu.make_async_copy(utab_hbm.at[pl.ds(0, BPW)], uspm.at[sid],
                          sem_u).wait()
    pltpu.make_async_copy(itab_hbm.at[pl.ds(0, BPW)], ispm.at[sid],
                          sem_i).wait()

    chunk_sems = ((sem_cu0, sem_ci0), (sem_cu1, sem_ci1))

    def load_chunk(c):
      buf = c % 2
      su, si = chunk_sems[buf]
      pltpu.async_copy(uspm.at[sid, pl.ds(c * CH, CH)], ubuf_v.at[buf], su)
      pltpu.async_copy(ispm.at[sid, pl.ds(c * CH, CH)], ibuf_v.at[buf], si)

    def wait_chunk(c):
      buf = c % 2
      su, si = chunk_sems[buf]
      pltpu.make_async_copy(uspm.at[sid, pl.ds(0, CH)], ubuf_v.at[buf],
                            su).wait()
      pltpu.make_async_copy(ispm.at[sid, pl.ds(0, CH)], ibuf_v.at[buf],
                            si).wait()

    def compute_group(c, g, buf):
      res = bias
      for j in range(L):
        r = g * L + j
        p = (ubuf_v[buf, r, pl.ds(0, L)] * ibuf_v[buf, r, pl.ds(0, L)] * w0
             + ubuf_v[buf, r, pl.ds(L, L)] * ibuf_v[buf, r, pl.ds(L, L)]
             * w1)
        s = jnp.sum(p)
        res = jnp.where(lane == j, res + s, res)
      out_v[pl.ds(c * CH + g * L, L)] = 1.0 / (1.0 + jnp.exp(-res))

    load_chunk(0)
    for c in range(NCHUNK):
      if c + 1 < NCHUNK:
        load_chunk(c + 1)
      wait_chunk(c)
      for g in range(CH // L):
        compute_group(c, g, c % 2)
    pltpu.sync_copy(out_v, out_hbm.at[pl.ds(base, BPW)])

  return sc_kernel


_SC_KERNEL = None


def kernel(user, item, user_table, item_table, W, b):
  global _SC_KERNEL
  if _SC_KERNEL is None:
    _SC_KERNEL = _make_sc_kernel()
  wb = jnp.concatenate([
      W.reshape(DIM).astype(jnp.float32),
      jnp.broadcast_to(b.astype(jnp.float32), (L,)),
  ])
  score = _SC_KERNEL(user.astype(jnp.int32), item.astype(jnp.int32), wb,
                     user_table, item_table)
  return score.reshape(BATCH, 1)


# speedup vs baseline: 1.0950x; 1.0950x over previous
import functools

import jax
import jax.numpy as jnp
from jax import lax
from jax.experimental import pallas as pl
from jax.experimental.pallas import tpu as pltpu
from jax.experimental.pallas import tpu_sc as plsc

NC = 2
NS = 16
L = 16
NW = NC * NS

BATCH = 16384
DIM = 32
BPW = BATCH // NW
NG = BPW // L


def _make_sc_kernel():
  mesh = plsc.VectorSubcoreMesh(core_axis_name="c", subcore_axis_name="s",
                                num_cores=NC, num_subcores=NS)

  @functools.partial(
      pl.kernel,
      out_type=jax.ShapeDtypeStruct((BATCH,), jnp.float32),
      mesh=mesh,
      scratch_types=[
          pltpu.VMEM((BPW,), jnp.int32),
          pltpu.VMEM((BPW,), jnp.int32),
          pltpu.VMEM((2, L, DIM), jnp.float32),
          pltpu.VMEM((2, L, DIM), jnp.float32),
          pltpu.VMEM((48,), jnp.float32),
          pltpu.VMEM((BPW,), jnp.float32),
          pltpu.SemaphoreType.DMA,
          pltpu.SemaphoreType.DMA,
      ],
      compiler_params=pltpu.CompilerParams(needs_layout_passes=False),
  )
  def sc_kernel(user_hbm, item_hbm, wb_hbm, utab_hbm, itab_hbm, out_hbm,
                uidx_v, iidx_v, ubuf_v, ibuf_v, wb_v, out_v, sem_u, sem_i):
    wid = lax.axis_index("s") * NC + lax.axis_index("c")
    base = wid * BPW

    pltpu.sync_copy(wb_hbm, wb_v)
    pltpu.sync_copy(user_hbm.at[pl.ds(base, BPW)], uidx_v)
    pltpu.sync_copy(item_hbm.at[pl.ds(base, BPW)], iidx_v)

    lane = lax.iota(jnp.int32, L)
    bias = wb_v[pl.ds(DIM, L)]
    w0 = wb_v[pl.ds(0, L)]
    w1 = wb_v[pl.ds(L, L)]

    def fire(g, buf):
      uvec = uidx_v[pl.ds(g * L, L)]
      ivec = iidx_v[pl.ds(g * L, L)]
      for j in range(L):
        pltpu.async_copy(utab_hbm.at[pl.ds(uvec[j], 1)],
                         ubuf_v.at[buf, pl.ds(j, 1)], sem_u)
        pltpu.async_copy(itab_hbm.at[pl.ds(ivec[j], 1)],
                         ibuf_v.at[buf, pl.ds(j, 1)], sem_i)

    def drain(buf):
      pltpu.make_async_copy(utab_hbm.at[pl.ds(0, L)], ubuf_v.at[buf],
                            sem_u).wait()
      pltpu.make_async_copy(itab_hbm.at[pl.ds(0, L)], ibuf_v.at[buf],
                            sem_i).wait()

    def compute(g, buf):
      res = bias
      for j in range(L):
        p = (ubuf_v[buf, j, pl.ds(0, L)] * ibuf_v[buf, j, pl.ds(0, L)] * w0
             + ubuf_v[buf, j, pl.ds(L, L)] * ibuf_v[buf, j, pl.ds(L, L)] * w1)
        s = jnp.sum(p)
        res = jnp.where(lane == j, res + s, res)
      out_v[pl.ds(g * L, L)] = 1.0 / (1.0 + jnp.exp(-res))

    fire(0, 0)

    def step(g, carry):
      buf = lax.rem(g, 2)
      nbuf = 1 - buf

      @pl.when(g + 1 < NG)
      def _():
        fire(g + 1, nbuf)

      drain(buf)
      compute(g, buf)
      return carry

    lax.fori_loop(0, NG, step, 0)
    pltpu.sync_copy(out_v, out_hbm.at[pl.ds(base, BPW)])

  return sc_kernel


_SC_KERNEL = None


def kernel(user, item, user_table, item_table, W, b):
  global _SC_KERNEL
  if _SC_KERNEL is None:
    _SC_KERNEL = _make_sc_kernel()
  wb = jnp.concatenate([
      W.reshape(DIM).astype(jnp.float32),
      jnp.broadcast_to(b.astype(jnp.float32), (L,)),
  ])
  score = _SC_KERNEL(user.astype(jnp.int32), item.astype(jnp.int32), wb,
                     user_table, item_table)
  return score.reshape(BATCH, 1)
